# baseline (speedup 1.0000x reference)
import jax
import jax.numpy as jnp
from jax import lax
from jax.experimental import pallas as pl
from jax.experimental.pallas import tpu as pltpu
from jax.experimental.pallas import tpu_sc as plsc

_SB, _B, _NR, _NPTS = 8, 4096, 8, 100000
_NC, _NS, _L = 2, 16, 16
_NW = _NC * _NS
_CPS = _NW // _SB
_CHUNK = _B // _CPS
_NSTEP = _CHUNK // _L
_NGRP = _NR * _CHUNK // 128
_GPR = _CHUNK // 128


def _sc_body(rgb_hbm, rows_hbm, color_hbm, mask_hbm, out_hbm,
             rows_v, rgb_v, col_v, msk_v, acc_v, sem):
    wid = lax.axis_index("s") * _NC + lax.axis_index("c")
    scene = wid // _CPS
    n0 = (wid % _CPS) * _CHUNK

    row_copies = []
    for r in range(_NR):
        src0 = (scene * _NR + r) * (_B // 128) + n0 // 128
        row_copies.append(
            pltpu.async_copy(rows_hbm.at[pl.ds(src0, _GPR)],
                             rows_v.at[pl.ds(r * _GPR, _GPR)], sem))
    rgb_copy = pltpu.async_copy(
        rgb_hbm.at[scene, pl.ds(n0 * _NR * 3, _CHUNK * _NR * 3)], rgb_v, sem)
    for c in row_copies:
        c.wait()
    rgb_copy.wait()

    def fire_body(g, _):
        c = pltpu.async_copy(color_hbm.at[rows_v.at[g]],
                             col_v.at[pl.ds(g * 128, 128)], sem)
        m = pltpu.async_copy(mask_hbm.at[rows_v.at[g]],
                             msk_v.at[pl.ds(g * 128, 128)], sem)
        c.wait()
        m.wait()
        return 0

    lax.fori_loop(0, _NGRP, fire_body, 0)

    iota = lax.broadcasted_iota(jnp.int32, (_L,), 0)
    zero = jnp.zeros((_L,), jnp.float32)
    acc_v[0, :] = zero
    acc_v[1, :] = zero

    def ray_step(r, s):
        ids = s * _L + iota
        rid = r * _CHUNK + ids
        fbase = ids * (_NR * 3) + r * 3
        d = zero
        for ch in range(3):
            rv = plsc.load_gather(rgb_v, [fbase + ch])
            cv = plsc.load_gather(col_v, [rid, jnp.full((_L,), ch, jnp.int32)])
            d = d + jnp.abs(rv - cv)
        m = msk_v[pl.ds(r * _CHUNK + s * _L, _L)]
        acc_v[0, :] = acc_v[0, :] + d * m
        acc_v[1, :] = acc_v[1, :] + m

    for r in range(_NR):
        def body(s, _, r=r):
            ray_step(r, s)
            return 0

        lax.fori_loop(0, _NSTEP, body, 0)

    plsc.subcore_barrier()
    pltpu.sync_copy(acc_v, out_hbm.at[wid])


def _make_sc_kernel(interpret=False):
    return pl.kernel(
        _sc_body,
        out_type=jax.ShapeDtypeStruct((_NW, 2, _L), jnp.float32),
        mesh=plsc.VectorSubcoreMesh(core_axis_name="c", subcore_axis_name="s"),
        compiler_params=pltpu.CompilerParams(needs_layout_passes=False,
                                             use_tc_tiling_on_sc=False),
        scratch_types=[
            pltpu.VMEM((_NGRP, 128), jnp.int32),
            pltpu.VMEM((_CHUNK * _NR * 3,), jnp.float32),
            pltpu.VMEM((_NR * _CHUNK, 3), jnp.float32),
            pltpu.VMEM((_NR * _CHUNK,), jnp.float32),
            pltpu.VMEM((2, _L), jnp.float32),
            pltpu.SemaphoreType.DMA,
        ],
        interpret=interpret,
    )


_rgbref_loss_sc = _make_sc_kernel()


def kernel(rgb_ref, idx_pcloud, color_pcloud, mask_pcloud):
    rgb = rgb_ref.reshape(_SB, _B * _NR * 3)
    base = (jnp.arange(_SB * _NR, dtype=jnp.int32) * _NPTS).reshape(_SB, _NR, 1)
    rows = (base + idx_pcloud.reshape(_SB, 1, _B)).reshape(-1, 128)
    color = color_pcloud.reshape(_SB * _NR * _NPTS, 3)
    mask = mask_pcloud.reshape(_SB * _NR * _NPTS)

    parts = _rgbref_loss_sc(rgb, rows, color, mask)
    parts = parts.reshape(_SB, _CPS, 2, _L)
    abs_sum = parts[:, :, 0, :].sum(axis=(1, 2))
    mask_sum = parts[:, :, 1, :].sum(axis=(1, 2))
    loss = jnp.where(mask_sum > 0,
                     abs_sum / jnp.maximum(mask_sum * 3.0, 1e-12),
                     mask_sum)
    return jnp.mean(loss)


# speedup vs baseline: 1.0875x; 1.0875x over previous
import jax
import jax.numpy as jnp
from jax import lax
from jax.experimental import pallas as pl
from jax.experimental.pallas import tpu as pltpu
from jax.experimental.pallas import tpu_sc as plsc

_SB, _B, _NR, _NPTS = 8, 4096, 8, 100000
_NC, _NS, _L = 2, 16, 16
_NW = _NC * _NS
_CPS = _NW // _SB
_CHUNK = _B // _CPS
_GPR = _CHUNK // 128
_NSEG = _NR * 3
_CGRP = _NSEG * _GPR
_MGRP = _NR * _GPR


def _sc_body(rgbt_hbm, crows_hbm, mrows_hbm, color_hbm, mask_hbm, out_hbm,
             crows_v, mrows_v, rgb_v, col_v, msk_v, acc_v, sem):
    wid = lax.axis_index("s") * _NC + lax.axis_index("c")
    scene = wid // _CPS
    n0 = (wid % _CPS) * _CHUNK

    stage = []
    for seg in range(_NSEG):
        off = (scene * _NSEG + seg) * _B + n0
        stage.append(pltpu.async_copy(rgbt_hbm.at[pl.ds(off, _CHUNK)],
                                      rgb_v.at[pl.ds(seg * _CHUNK, _CHUNK)],
                                      sem))
    for c in stage:
        c.wait()

    def stage_cidx(b, _):
        descs = []
        for t in range(16):
            g = b * 16 + t
            off = (scene * _NSEG + g // _GPR) * _B + n0 + (g % _GPR) * 128
            descs.append(pltpu.async_copy(crows_hbm.at[pl.ds(off, 128)],
                                          crows_v.at[g], sem))
        for d in descs:
            d.wait()
        return 0

    def stage_midx(b, _):
        descs = []
        for t in range(16):
            g = b * 16 + t
            off = (scene * _NR + g // _GPR) * _B + n0 + (g % _GPR) * 128
            descs.append(pltpu.async_copy(mrows_hbm.at[pl.ds(off, 128)],
                                          mrows_v.at[g], sem))
        for d in descs:
            d.wait()
        return 0

    lax.fori_loop(0, _CGRP // 16, stage_cidx, 0)
    lax.fori_loop(0, _MGRP // 16, stage_midx, 0)

    def fire_color(b, _):
        descs = []
        for t in range(8):
            g = b * 8 + t
            descs.append(pltpu.async_copy(color_hbm.at[crows_v.at[g]],
                                          col_v.at[pl.ds(g * 128, 128)], sem))
        for d in descs:
            d.wait()
        return 0

    def fire_mask(b, _):
        descs = []
        for t in range(8):
            g = b * 8 + t
            descs.append(pltpu.async_copy(mask_hbm.at[mrows_v.at[g]],
                                          msk_v.at[pl.ds(g * 128, 128)], sem))
        for d in descs:
            d.wait()
        return 0

    lax.fori_loop(0, _CGRP // 8, fire_color, 0)
    lax.fori_loop(0, _MGRP // 8, fire_mask, 0)

    zero = jnp.zeros((_L,), jnp.float32)
    acc_v[0, :] = zero
    acc_v[1, :] = zero

    def step(r, s):
        m = msk_v[pl.ds(r * _CHUNK + s * _L, _L)]
        d = zero
        for ch in range(3):
            seg = (r * 3 + ch) * _CHUNK + s * _L
            d = d + jnp.abs(rgb_v[pl.ds(seg, _L)] - col_v[pl.ds(seg, _L)])
        acc_v[0, :] = acc_v[0, :] + d * m
        acc_v[1, :] = acc_v[1, :] + m

    for r in range(_NR):
        def body(s, _, r=r):
            step(r, s)
            return 0

        lax.fori_loop(0, _CHUNK // _L, body, 0)

    pltpu.sync_copy(acc_v, out_hbm.at[wid])


def _make_sc_kernel(interpret=False):
    return pl.kernel(
        _sc_body,
        out_type=jax.ShapeDtypeStruct((_NW, 2, _L), jnp.float32),
        mesh=plsc.VectorSubcoreMesh(core_axis_name="c", subcore_axis_name="s"),
        compiler_params=pltpu.CompilerParams(use_tc_tiling_on_sc=False),
        scratch_types=[
            pltpu.VMEM((_CGRP, 128), jnp.int32),
            pltpu.VMEM((_MGRP, 128), jnp.int32),
            pltpu.VMEM((_NSEG * _CHUNK,), jnp.float32),
            pltpu.VMEM((_NSEG * _CHUNK,), jnp.float32),
            pltpu.VMEM((_NR * _CHUNK,), jnp.float32),
            pltpu.VMEM((2, _L), jnp.float32),
            pltpu.SemaphoreType.DMA,
        ],
        interpret=interpret,
    )


_rgbref_loss_sc = _make_sc_kernel()


def kernel(rgb_ref, idx_pcloud, color_pcloud, mask_pcloud):
    idx = idx_pcloud.reshape(_SB, _B)

    rgbt = jnp.transpose(rgb_ref, (0, 2, 3, 1)).reshape(-1)

    sr = jnp.arange(_SB * _NR, dtype=jnp.int32).reshape(_SB, _NR) * _NPTS
    mrows = sr[:, :, None] + idx[:, None, :]
    crows = (mrows[:, :, None, :] * 3
             + jnp.arange(3, dtype=jnp.int32)[None, None, :, None])
    crows = crows.reshape(-1)
    mrows = mrows.reshape(-1)

    color = color_pcloud.reshape(-1)
    mask = mask_pcloud.reshape(-1)

    parts = _rgbref_loss_sc(rgbt, crows, mrows, color, mask)
    parts = parts.reshape(_SB, _CPS, 2, _L)
    abs_sum = parts[:, :, 0, :].sum(axis=(1, 2))
    mask_sum = parts[:, :, 1, :].sum(axis=(1, 2))
    loss = jnp.where(mask_sum > 0,
                     abs_sum / jnp.maximum(mask_sum * 3.0, 1e-12),
                     mask_sum)
    return jnp.mean(loss)
